# baseline (speedup 1.0000x reference)
import functools

import jax
import jax.numpy as jnp
from jax import lax
from jax.experimental import pallas as pl
from jax.experimental.pallas import tpu as pltpu
from jax.experimental.pallas import tpu_sc as plsc

B = 32
N = 1024
D = 128
V = 2048

CBLK = 1024
NCH = 8
CH = 128


def _tc_body(x_ref, adj_ref, idx_ref):
    cb = pl.program_id(1)
    xb = x_ref[0]
    c0 = cb * CBLK
    row = lax.broadcasted_iota(jnp.int32, (N, CBLK), 0)
    col = lax.broadcasted_iota(jnp.int32, (N, CBLK), 1) + c0
    adj_ref[0] = xb + (row == col).astype(jnp.float32)
    total = None
    for w in range(4):
        acc = xb[w * 256:w * 256 + 8]
        for k in range(1, 32):
            acc = acc + xb[w * 256 + 8 * k:w * 256 + 8 * k + 8]
        t = acc[0:4] + acc[4:8]
        u = t[0:2] + t[2:4]
        v = u[0:1] + u[1:2]
        total = v if total is None else total + v
    idx_ref[0] = total.astype(jnp.int32)


_tc_call = pl.pallas_call(
    _tc_body,
    grid=(B, N // CBLK),
    in_specs=[
        pl.BlockSpec((1, N, CBLK), lambda b, cb: (b, 0, cb)),
    ],
    out_specs=[
        pl.BlockSpec((1, N, CBLK), lambda b, cb: (b, 0, cb)),
        pl.BlockSpec((1, 1, CBLK), lambda b, cb: (b, 0, cb)),
    ],
    out_shape=[
        jax.ShapeDtypeStruct((B, N, N), jnp.float32),
        jax.ShapeDtypeStruct((B, 1, N), jnp.int32),
    ],
)


_sc_mesh = plsc.VectorSubcoreMesh(core_axis_name="c", subcore_axis_name="s")


@functools.partial(
    pl.kernel,
    mesh=_sc_mesh,
    compiler_params=pltpu.CompilerParams(needs_layout_passes=False),
    out_type=jax.ShapeDtypeStruct((B, D, N), jnp.float32),
    scratch_types=[
        pltpu.VMEM((NCH, CH), jnp.int32),
        pltpu.VMEM((2, CH, D), jnp.float32),
        pltpu.VMEM((2, D, CH), jnp.float32),
        pltpu.VMEM_SHARED((V, D), jnp.float32),
        pltpu.SemaphoreType.DMA,
        pltpu.SemaphoreType.DMA,
    ],
)
def _sc_gather_t(table_hbm, idx_hbm, out_hbm, idx_v, rows_v, outT_v,
                 table_sp, gsem, wsem):
    sid = lax.axis_index("s")
    wid = sid * 2 + lax.axis_index("c")
    b = wid
    vs = V // 16
    pltpu.sync_copy(
        table_hbm.at[pl.ds(sid * vs, vs)], table_sp.at[pl.ds(sid * vs, vs)]
    )
    pltpu.sync_copy(idx_hbm.at[b], idx_v)
    plsc.subcore_barrier()

    lanes = lax.iota(jnp.int32, 16)
    drows = [d0 + lanes for d0 in range(0, D, 16)]

    gh = [None, None]
    wh = [None, None]
    gh[0] = pltpu.async_copy(table_sp.at[idx_v.at[0]], rows_v.at[0], gsem)
    for c in range(NCH):
        j = c & 1
        if c + 1 < NCH:
            jn = (c + 1) & 1
            gh[jn] = pltpu.async_copy(
                table_sp.at[idx_v.at[c + 1]], rows_v.at[jn], gsem
            )
        gh[j].wait()
        if wh[j] is not None:
            wh[j].wait()

        @plsc.parallel_loop(0, CH, unroll=8)
        def _tr(n, _j=j):
            ncol = lanes * 0 + n
            for k in range(D // 16):
                v = rows_v[_j, n, pl.ds(k * 16, 16)]
                plsc.store_scatter(outT_v.at[_j], [drows[k], ncol], v)

        wh[j] = pltpu.async_copy(
            outT_v.at[j], out_hbm.at[b, :, pl.ds(c * CH, CH)], wsem
        )
    for h in wh:
        h.wait()


def kernel(x, table):
    adj, idx = _tc_call(x)
    idx = idx.reshape(B, NCH, CH)
    embed = _sc_gather_t(table, idx)
    return adj, embed


# speedup vs baseline: 1.2500x; 1.2500x over previous
import functools

import jax
import jax.numpy as jnp
from jax import lax
from jax.experimental import pallas as pl
from jax.experimental.pallas import tpu as pltpu
from jax.experimental.pallas import tpu_sc as plsc

B = 32
N = 1024
D = 128
V = 2048

CBLK = 1024
NCH = 8
CH = 128
NBUF = 4


def _tc_body(x_ref, adj_ref, idx_ref):
    cb = pl.program_id(1)
    xb = x_ref[0]
    c0 = cb * CBLK
    row = lax.broadcasted_iota(jnp.int32, (N, CBLK), 0)
    col = lax.broadcasted_iota(jnp.int32, (N, CBLK), 1) + c0
    adj_ref[0] = xb + (row == col).astype(jnp.float32)
    total = None
    for w in range(4):
        acc = xb[w * 256:w * 256 + 8]
        for k in range(1, 32):
            acc = acc + xb[w * 256 + 8 * k:w * 256 + 8 * k + 8]
        t = acc[0:4] + acc[4:8]
        u = t[0:2] + t[2:4]
        v = u[0:1] + u[1:2]
        total = v if total is None else total + v
    idx_ref[0] = total.astype(jnp.int32)


_tc_call = pl.pallas_call(
    _tc_body,
    grid=(B, N // CBLK),
    in_specs=[
        pl.BlockSpec((1, N, CBLK), lambda b, cb: (b, 0, cb)),
    ],
    out_specs=[
        pl.BlockSpec((1, N, CBLK), lambda b, cb: (b, 0, cb)),
        pl.BlockSpec((1, 1, CBLK), lambda b, cb: (b, 0, cb)),
    ],
    out_shape=[
        jax.ShapeDtypeStruct((B, N, N), jnp.float32),
        jax.ShapeDtypeStruct((B, 1, N), jnp.int32),
    ],
)


_sc_mesh = plsc.VectorSubcoreMesh(core_axis_name="c", subcore_axis_name="s")


@functools.partial(
    pl.kernel,
    mesh=_sc_mesh,
    compiler_params=pltpu.CompilerParams(needs_layout_passes=False),
    out_type=jax.ShapeDtypeStruct((B, N, D), jnp.float32),
    scratch_types=[
        pltpu.VMEM((NCH, CH), jnp.int32),
        pltpu.VMEM((NBUF, CH, D), jnp.float32),
        pltpu.VMEM_SHARED((V, D), jnp.float32),
        pltpu.SemaphoreType.DMA,
        pltpu.SemaphoreType.DMA,
    ],
)
def _sc_gather(table_hbm, idx_hbm, out_hbm, idx_v, rows_v, table_sp, gsem, wsem):
    sid = lax.axis_index("s")
    wid = sid * 2 + lax.axis_index("c")
    b = wid
    vs = V // 16
    pltpu.sync_copy(
        table_hbm.at[pl.ds(sid * vs, vs)], table_sp.at[pl.ds(sid * vs, vs)]
    )
    pltpu.sync_copy(idx_hbm.at[b], idx_v)
    plsc.subcore_barrier()
    prev_writes = []
    for g in range(NCH // NBUF):
        gathers = []
        for j in range(NBUF):
            c = g * NBUF + j
            if prev_writes:
                prev_writes[j].wait()
            gathers.append(
                pltpu.async_copy(table_sp.at[idx_v.at[c]], rows_v.at[j], gsem)
            )
        writes = []
        for j in range(NBUF):
            c = g * NBUF + j
            gathers[j].wait()
            writes.append(
                pltpu.async_copy(
                    rows_v.at[j], out_hbm.at[b, pl.ds(c * CH, CH)], wsem
                )
            )
        prev_writes = writes
    for wcopy in prev_writes:
        wcopy.wait()


def _tr_body(g_ref, out_ref):
    out_ref[0] = g_ref[0].T
    out_ref[1] = g_ref[1].T


_tr_call = pl.pallas_call(
    _tr_body,
    grid=(B // 2,),
    in_specs=[pl.BlockSpec((2, N, D), lambda b: (b, 0, 0))],
    out_specs=pl.BlockSpec((2, D, N), lambda b: (b, 0, 0)),
    out_shape=jax.ShapeDtypeStruct((B, D, N), jnp.float32),
    compiler_params=pltpu.CompilerParams(dimension_semantics=("parallel",)),
)


def kernel(x, table):
    adj, idx = _tc_call(x)
    idx = idx.reshape(B, NCH, CH)
    rows = _sc_gather(table, idx)
    embed = _tr_call(rows)
    return adj, embed


# speedup vs baseline: 1.2951x; 1.0361x over previous
import functools

import jax
import jax.numpy as jnp
from jax import lax
from jax.experimental import pallas as pl
from jax.experimental.pallas import tpu as pltpu
from jax.experimental.pallas import tpu_sc as plsc

B = 32
N = 1024
D = 128
V = 2048

CBLK = 1024
NCH = 8
CH = 128
NBUF = 4


def _tc_body(x_ref, adj_ref, idx_ref):
    cb = pl.program_id(1)
    xb = x_ref[0]
    c0 = cb * CBLK
    row = lax.broadcasted_iota(jnp.int32, (N, CBLK), 0)
    col = lax.broadcasted_iota(jnp.int32, (N, CBLK), 1) + c0
    adj_ref[0] = xb + (row == col).astype(jnp.float32)
    total = None
    for w in range(4):
        acc = xb[w * 256:w * 256 + 8]
        for k in range(1, 32):
            acc = acc + xb[w * 256 + 8 * k:w * 256 + 8 * k + 8]
        t = acc[0:4] + acc[4:8]
        u = t[0:2] + t[2:4]
        v = u[0:1] + u[1:2]
        total = v if total is None else total + v
    idx_ref[0] = total.astype(jnp.int32)


_tc_call = pl.pallas_call(
    _tc_body,
    grid=(B, N // CBLK),
    in_specs=[
        pl.BlockSpec((1, N, CBLK), lambda b, cb: (b, 0, cb)),
    ],
    out_specs=[
        pl.BlockSpec((1, N, CBLK), lambda b, cb: (b, 0, cb)),
        pl.BlockSpec((1, 1, CBLK), lambda b, cb: (b, 0, cb)),
    ],
    out_shape=[
        jax.ShapeDtypeStruct((B, N, N), jnp.float32),
        jax.ShapeDtypeStruct((B, 1, N), jnp.int32),
    ],
)


_sc_mesh = plsc.VectorSubcoreMesh(core_axis_name="c", subcore_axis_name="s")


@functools.partial(
    pl.kernel,
    mesh=_sc_mesh,
    compiler_params=pltpu.CompilerParams(needs_layout_passes=False),
    out_type=jax.ShapeDtypeStruct((B, N, D), jnp.float32),
    scratch_types=[
        pltpu.VMEM((NCH, CH), jnp.int32),
        pltpu.VMEM((NBUF, CH, D), jnp.float32),
        pltpu.VMEM_SHARED((V, D), jnp.float32),
        pltpu.SemaphoreType.DMA,
        pltpu.SemaphoreType.DMA,
    ],
)
def _sc_gather(table_hbm, idx_hbm, out_hbm, idx_v, rows_v, table_sp, gsem, wsem):
    sid = lax.axis_index("s")
    wid = sid * 2 + lax.axis_index("c")
    b = wid
    vs = V // 16
    pltpu.sync_copy(
        table_hbm.at[pl.ds(sid * vs, vs)], table_sp.at[pl.ds(sid * vs, vs)]
    )
    pltpu.sync_copy(idx_hbm.at[b], idx_v)
    plsc.subcore_barrier()
    prev_writes = []
    for g in range(NCH // NBUF):
        gathers = []
        for j in range(NBUF):
            c = g * NBUF + j
            if prev_writes:
                prev_writes[j].wait()
            gathers.append(
                pltpu.async_copy(table_sp.at[idx_v.at[c]], rows_v.at[j], gsem)
            )
        writes = []
        for j in range(NBUF):
            c = g * NBUF + j
            gathers[j].wait()
            writes.append(
                pltpu.async_copy(
                    rows_v.at[j], out_hbm.at[b, pl.ds(c * CH, CH)], wsem
                )
            )
        prev_writes = writes
    for wcopy in prev_writes:
        wcopy.wait()


def _tr_body(g_ref, out_ref):
    for i in range(4):
        out_ref[i] = g_ref[i].T


_tr_call = pl.pallas_call(
    _tr_body,
    grid=(B // 4,),
    in_specs=[pl.BlockSpec((4, N, D), lambda b: (b, 0, 0))],
    out_specs=pl.BlockSpec((4, D, N), lambda b: (b, 0, 0)),
    out_shape=jax.ShapeDtypeStruct((B, D, N), jnp.float32),
    compiler_params=pltpu.CompilerParams(dimension_semantics=("parallel",)),
)


def kernel(x, table):
    adj, idx = _tc_call(x)
    idx = idx.reshape(B, NCH, CH)
    rows = _sc_gather(table, idx)
    embed = _tr_call(rows)
    return adj, embed
